# Initial kernel scaffold; baseline (speedup 1.0000x reference)
import jax
import jax.numpy as jnp
from jax.experimental import pallas as pl


def kernel(ls, positions, values, pn_w, db_down0_w, coarsen0_w, db_down1_w, coarsen1_w, bottleneck_w, finefy0_w, db_up0_w, finefy1_w, db_up1_w, stepdown_w):
    raise NotImplementedError("write your pallas kernel here")


# speedup vs baseline: 3.6081x; 3.6081x over previous
import functools

import jax
import jax.numpy as jnp
from jax import lax
from jax.experimental import pallas as pl
from jax.experimental.pallas import tpu as pltpu
from jax.experimental.pallas import tpu_sc as plsc

N = 100000
M = 50000
NCLS = 21

NC = 2
NS = 16
NW = NC * NS
CHUNK = 128
PPW = 3200
KCH = PPW // CHUNK
NPAD = NW * PPW
D = 16
MT = 51200
MPW = MT // NS
ZROWS = 640

def _splat_body(pts_hbm, idx_hbm, out_hbm, idx_v, rows_v, zbuf, acc):
    cid = lax.axis_index("c")
    sid = lax.axis_index("s")
    wid = sid * NC + cid

    def _zf(i, c):
        zbuf[i, :] = jnp.zeros((16,), jnp.float32)
        return c
    lax.fori_loop(0, ZROWS, _zf, 0)
    for j in range(MPW // ZROWS):
        pltpu.sync_copy(zbuf, acc.at[pl.ds(sid * MPW + j * ZROWS, ZROWS)])
    plsc.subcore_barrier()

    base = wid * PPW
    pltpu.sync_copy(pts_hbm.at[pl.ds(base, PPW)], rows_v)
    pltpu.sync_copy(idx_hbm.at[wid], idx_v)

    def _sc(j, c):
        pltpu.sync_copy(rows_v.at[pl.ds(j * CHUNK, CHUNK)],
                        acc.at[idx_v.at[j]], add=True)
        return c
    lax.fori_loop(0, KCH, _sc, 0)
    plsc.subcore_barrier()

    pltpu.sync_copy(acc.at[pl.ds(sid * MPW, MPW)],
                    out_hbm.at[pl.ds(cid * MT + sid * MPW, MPW)])


@functools.cache
def _sc_kernels():
    mesh = plsc.VectorSubcoreMesh(core_axis_name="c", subcore_axis_name="s",
                                  num_cores=NC, num_subcores=NS)
    params = pltpu.CompilerParams(use_tc_tiling_on_sc=False)
    splat = pl.kernel(
        _splat_body,
        out_type=jax.ShapeDtypeStruct((NC * MT, D), jnp.float32),
        mesh=mesh,
        scratch_types=[
            pltpu.VMEM((KCH, CHUNK), jnp.int32),
            pltpu.VMEM((PPW, D), jnp.float32),
            pltpu.VMEM((ZROWS, D), jnp.float32),
            pltpu.VMEM_SHARED((MT, D), jnp.float32),
        ],
        compiler_params=params,
    )
    slice_k = pl.kernel(
        _slice_body,
        out_type=[jax.ShapeDtypeStruct((NPAD, D), jnp.float32),
                  jax.ShapeDtypeStruct((NPAD, D), jnp.float32)],
        mesh=mesh,
        scratch_types=[
            pltpu.VMEM((KCH, CHUNK), jnp.int32),
            pltpu.VMEM((KCH, CHUNK), jnp.int32),
            pltpu.VMEM((CHUNK, D), jnp.float32),
            pltpu.VMEM((CHUNK, D), jnp.float32),
            pltpu.SemaphoreType.DMA,
            pltpu.SemaphoreType.DMA,
        ],
        compiler_params=params,
    )
    return splat, slice_k


def _slice_body(a_hbm, b_hbm, idx1_hbm, idx2_hbm, g1_hbm, g2_hbm,
                idx1_v, idx2_v, bufa, bufb, sema, semb):
    cid = lax.axis_index("c")
    sid = lax.axis_index("s")
    wid = sid * NC + cid
    base = wid * PPW
    pltpu.sync_copy(idx1_hbm.at[wid], idx1_v)
    pltpu.sync_copy(idx2_hbm.at[wid], idx2_v)

    def _g(j, c):
        ca = pltpu.async_copy(a_hbm.at[idx1_v.at[j]], bufa, sema)
        cb = pltpu.async_copy(b_hbm.at[idx2_v.at[j]], bufb, semb)
        ca.wait()
        cb.wait()
        pltpu.sync_copy(bufa, g1_hbm.at[pl.ds(base + j * CHUNK, CHUNK)])
        pltpu.sync_copy(bufb, g2_hbm.at[pl.ds(base + j * CHUNK, CHUNK)])
        return c
    lax.fori_loop(0, KCH, _g, 0)


RB = 2000
GB = M // RB


def _relu(x):
    return jnp.maximum(x, 0.0)


def _mm(x, w):
    return jnp.dot(x, w, preferred_element_type=jnp.float32)


def _dense_grow(x, ws):
    for w in ws:
        x = jnp.concatenate([x, _relu(_mm(x, w))], axis=1)
    return x


def _dense_feats(x, ws):
    c0 = x.shape[1]
    return _dense_grow(x, ws)[:, c0:]


def _pairmean(y):
    r, c = y.shape
    y = y.reshape(r // 2, 2, c)
    return (y[:, 0, :] + y[:, 1, :]) * 0.5


def _rep2(y):
    r, c = y.shape
    return jnp.broadcast_to(y[:, None, :], (r, 2, c)).reshape(2 * r, c)


def _unet_body(*refs):
    it = iter(refs)
    sums_ref = next(it)
    pn = [next(it) for _ in range(3)]
    d0 = [next(it) for _ in range(4)]
    c0 = next(it)
    d1 = [next(it) for _ in range(5)]
    c1 = next(it)
    bn = [next(it) for _ in range(7)]
    f0 = next(it)
    u0 = [next(it) for _ in range(5)]
    f1 = next(it)
    u1 = [next(it) for _ in range(4)]
    sd0 = next(it)
    a_ref = next(it)
    b_ref = next(it)

    s = sums_ref[0] + sums_ref[1]
    cnt = s[:, 6:7]
    dvals = s[:, :6] / jnp.maximum(cnt, 1.0)
    h = dvals
    for w in pn:
        h = _relu(_mm(h, w[...]))
    x = _dense_grow(h, [w[...] for w in d0])
    fv0 = x
    x = _pairmean(_mm(_relu(x), c0[...]))
    x = _dense_grow(x, [w[...] for w in d1])
    fv1 = x
    x = _pairmean(_mm(_relu(x), c1[...]))
    blk = _dense_feats(x, [w[...] for w in bn])
    fine = _rep2(_mm(_relu(blk), f0[...]))
    x = jnp.concatenate([fine, fv1], axis=1)
    blk = _dense_feats(x, [w[...] for w in u0])
    fine = _rep2(_mm(_relu(blk), f1[...]))
    x = jnp.concatenate([fine, fv0], axis=1)
    blk = _dense_feats(x, [w[...] for w in u1])
    w0 = sd0[...]
    a_ref[...] = _mm(blk, w0[:32])
    b_ref[...] = _mm(blk, w0[32:])


HB = 4000
GH = N // HB


def _head_body(g1_ref, g2_ref, w1_ref, w2_ref, out_ref):
    h = _relu(g1_ref[...] + g2_ref[...])
    h = _relu(_mm(h, w1_ref[...]))
    lg = _mm(h, w2_ref[...])
    m = jnp.max(lg, axis=1, keepdims=True)
    e = jnp.exp(lg - m)
    out_ref[...] = lg - m - jnp.log(jnp.sum(e, axis=1, keepdims=True))


def _wspec(w):
    return pl.BlockSpec(w.shape, lambda i, _nd=w.ndim: (0,) * _nd)


def kernel(ls, positions, values, pn_w, db_down0_w, coarsen0_w, db_down1_w,
           coarsen1_w, bottleneck_w, finefy0_w, db_up0_w, finefy1_w,
           db_up1_w, stepdown_w):
    f32 = jnp.float32
    pts = jnp.concatenate(
        [positions[0], values[0],
         jnp.ones((N, 1), f32), jnp.zeros((N, D - 7), f32)], axis=1)
    pts_pad = jnp.concatenate([pts, jnp.zeros((NPAD - N, D), f32)], axis=0)
    ls_pad = jnp.concatenate([ls, jnp.zeros((NPAD - N,), jnp.int32)])
    ls3d = ls_pad.reshape(NW, KCH, CHUNK)
    lsn = (ls_pad + 1) % M
    lsn3d = lsn.reshape(NW, KCH, CHUNK)

    splat, slice_k = _sc_kernels()
    sums_flat = splat(pts_pad, ls3d)
    sums2 = sums_flat.reshape(NC, MT, D)

    ws = ([*pn_w, *db_down0_w, coarsen0_w, *db_down1_w, coarsen1_w,
           *bottleneck_w, finefy0_w, *db_up0_w, finefy1_w, *db_up1_w,
           stepdown_w[0]])
    a16, b16 = pl.pallas_call(
        _unet_body,
        grid=(GB,),
        in_specs=[pl.BlockSpec((NC, RB, D), lambda i: (0, i, 0))]
                 + [_wspec(w) for w in ws],
        out_specs=[pl.BlockSpec((RB, D), lambda i: (i, 0)),
                   pl.BlockSpec((RB, D), lambda i: (i, 0))],
        out_shape=[jax.ShapeDtypeStruct((M, D), f32),
                   jax.ShapeDtypeStruct((M, D), f32)],
        compiler_params=pltpu.CompilerParams(
            dimension_semantics=("arbitrary",)),
    )(sums2, *ws)

    g1, g2 = slice_k(a16, b16, ls3d, lsn3d)

    logits = pl.pallas_call(
        _head_body,
        grid=(GH,),
        in_specs=[pl.BlockSpec((HB, D), lambda i: (i, 0)),
                  pl.BlockSpec((HB, D), lambda i: (i, 0)),
                  _wspec(stepdown_w[1]), _wspec(stepdown_w[2])],
        out_specs=pl.BlockSpec((HB, NCLS), lambda i: (i, 0)),
        out_shape=jax.ShapeDtypeStruct((N, NCLS), f32),
        compiler_params=pltpu.CompilerParams(
            dimension_semantics=("arbitrary",)),
    )(g1[:N], g2[:N], stepdown_w[1], stepdown_w[2])

    return logits[None]


# speedup vs baseline: 4.5703x; 1.2667x over previous
import functools

import jax
import jax.numpy as jnp
from jax import lax
from jax.experimental import pallas as pl
from jax.experimental.pallas import tpu as pltpu
from jax.experimental.pallas import tpu_sc as plsc

N = 100000
M = 50000
NCLS = 21

NC = 2
NS = 16
NW = NC * NS
CHUNK = 128
PPW = 3200
KCH = PPW // CHUNK
NPAD = NW * PPW
D = 16
MT = 51200
MPW = MT // NS
ZROWS = 640

def _splat_body(pts_hbm, idx_hbm, out_hbm, idx_v, rows_v, zbuf, acc):
    cid = lax.axis_index("c")
    sid = lax.axis_index("s")
    wid = sid * NC + cid

    def _zf(i, c):
        zbuf[i, :] = jnp.zeros((16,), jnp.float32)
        return c
    lax.fori_loop(0, ZROWS, _zf, 0)
    for j in range(MPW // ZROWS):
        pltpu.sync_copy(zbuf, acc.at[pl.ds(sid * MPW + j * ZROWS, ZROWS)])
    plsc.subcore_barrier()

    base = wid * PPW
    pltpu.sync_copy(pts_hbm.at[pl.ds(base, PPW)], rows_v)
    pltpu.sync_copy(idx_hbm.at[wid], idx_v)

    def _sc(j, c):
        pltpu.sync_copy(rows_v.at[pl.ds(j * CHUNK, CHUNK)],
                        acc.at[idx_v.at[j]], add=True)
        return c
    lax.fori_loop(0, KCH, _sc, 0)
    plsc.subcore_barrier()

    pltpu.sync_copy(acc.at[pl.ds(sid * MPW, MPW)],
                    out_hbm.at[pl.ds(cid * MT + sid * MPW, MPW)])


@functools.cache
def _sc_kernels():
    mesh = plsc.VectorSubcoreMesh(core_axis_name="c", subcore_axis_name="s",
                                  num_cores=NC, num_subcores=NS)
    params = pltpu.CompilerParams(use_tc_tiling_on_sc=False)
    splat = pl.kernel(
        _splat_body,
        out_type=jax.ShapeDtypeStruct((NC * MT, D), jnp.float32),
        mesh=mesh,
        scratch_types=[
            pltpu.VMEM((KCH, CHUNK), jnp.int32),
            pltpu.VMEM((PPW, D), jnp.float32),
            pltpu.VMEM((ZROWS, D), jnp.float32),
            pltpu.VMEM_SHARED((MT, D), jnp.float32),
        ],
        compiler_params=params,
    )
    slice_k = pl.kernel(
        _slice_body,
        out_type=[jax.ShapeDtypeStruct((NPAD, D), jnp.float32),
                  jax.ShapeDtypeStruct((NPAD, D), jnp.float32)],
        mesh=mesh,
        scratch_types=[
            pltpu.VMEM((KCH, CHUNK), jnp.int32),
            pltpu.VMEM((KCH, CHUNK), jnp.int32),
            pltpu.VMEM((CHUNK, D), jnp.float32),
            pltpu.VMEM((CHUNK, D), jnp.float32),
            pltpu.SemaphoreType.DMA,
            pltpu.SemaphoreType.DMA,
        ],
        compiler_params=params,
    )
    return splat, slice_k


def _slice_body(a_hbm, b_hbm, idx1_hbm, idx2_hbm, g1_hbm, g2_hbm,
                idx1_v, idx2_v, bufa, bufb, sema, semb):
    cid = lax.axis_index("c")
    sid = lax.axis_index("s")
    wid = sid * NC + cid
    base = wid * PPW
    pltpu.sync_copy(idx1_hbm.at[wid], idx1_v)
    pltpu.sync_copy(idx2_hbm.at[wid], idx2_v)

    def _g(j, c):
        ca = pltpu.async_copy(a_hbm.at[idx1_v.at[j]], bufa, sema)
        cb = pltpu.async_copy(b_hbm.at[idx2_v.at[j]], bufb, semb)
        ca.wait()
        cb.wait()
        pltpu.sync_copy(bufa, g1_hbm.at[pl.ds(base + j * CHUNK, CHUNK)])
        pltpu.sync_copy(bufb, g2_hbm.at[pl.ds(base + j * CHUNK, CHUNK)])
        return c
    lax.fori_loop(0, KCH, _g, 0)


RB = 1600
GB = MT // RB


def _relu(x):
    return jnp.maximum(x, 0.0)


def _mm(x, w):
    return jnp.dot(x, w, preferred_element_type=jnp.float32)


def _dense_grow(x, ws):
    for w in ws:
        x = jnp.concatenate([x, _relu(_mm(x, w))], axis=1)
    return x


def _dense_feats(x, ws):
    c0 = x.shape[1]
    return _dense_grow(x, ws)[:, c0:]


def _pairmean(y):
    r, c = y.shape
    y = y.reshape(r // 2, 2, c)
    return (y[:, 0, :] + y[:, 1, :]) * 0.5


def _rep2(y):
    r, c = y.shape
    return jnp.broadcast_to(y[:, None, :], (r, 2, c)).reshape(2 * r, c)


def _unet_body(*refs):
    it = iter(refs)
    sums0_ref = next(it)
    sums1_ref = next(it)
    pn = [next(it) for _ in range(3)]
    d0 = [next(it) for _ in range(4)]
    c0 = next(it)
    d1 = [next(it) for _ in range(5)]
    c1 = next(it)
    bn = [next(it) for _ in range(7)]
    f0 = next(it)
    u0 = [next(it) for _ in range(5)]
    f1 = next(it)
    u1 = [next(it) for _ in range(4)]
    sd0 = next(it)
    a_ref = next(it)
    b_ref = next(it)

    s = sums0_ref[...] + sums1_ref[...]
    cnt = s[:, 6:7]
    dvals = s[:, :6] / jnp.maximum(cnt, 1.0)
    h = dvals
    for w in pn:
        h = _relu(_mm(h, w[...]))
    x = _dense_grow(h, [w[...] for w in d0])
    fv0 = x
    x = _pairmean(_mm(_relu(x), c0[...]))
    x = _dense_grow(x, [w[...] for w in d1])
    fv1 = x
    x = _pairmean(_mm(_relu(x), c1[...]))
    blk = _dense_feats(x, [w[...] for w in bn])
    fine = _rep2(_mm(_relu(blk), f0[...]))
    x = jnp.concatenate([fine, fv1], axis=1)
    blk = _dense_feats(x, [w[...] for w in u0])
    fine = _rep2(_mm(_relu(blk), f1[...]))
    x = jnp.concatenate([fine, fv0], axis=1)
    blk = _dense_feats(x, [w[...] for w in u1])
    w0 = sd0[...]
    a_ref[...] = _mm(blk, w0[:32])
    b_ref[...] = _mm(blk, w0[32:])


HB = 4000
GH = N // HB


def _head_body(g1_ref, g2_ref, w1_ref, w2_ref, out_ref):
    h = _relu(g1_ref[...] + g2_ref[...])
    h = _relu(_mm(h, w1_ref[...]))
    lg = _mm(h, w2_ref[...])
    m = jnp.max(lg, axis=1, keepdims=True)
    e = jnp.exp(lg - m)
    out_ref[0] = lg - m - jnp.log(jnp.sum(e, axis=1, keepdims=True))


def _wspec(w):
    return pl.BlockSpec(w.shape, lambda i, _nd=w.ndim: (0,) * _nd)


def kernel(ls, positions, values, pn_w, db_down0_w, coarsen0_w, db_down1_w,
           coarsen1_w, bottleneck_w, finefy0_w, db_up0_w, finefy1_w,
           db_up1_w, stepdown_w):
    f32 = jnp.float32
    pts = jnp.concatenate(
        [positions[0], values[0],
         jnp.ones((N, 1), f32), jnp.zeros((N, D - 7), f32)], axis=1)
    pts_pad = jnp.concatenate([pts, jnp.zeros((NPAD - N, D), f32)], axis=0)
    ls_pad = jnp.concatenate([ls, jnp.zeros((NPAD - N,), jnp.int32)])
    ls3d = ls_pad.reshape(NW, KCH, CHUNK)
    lsn = (ls_pad + 1) % M
    lsn3d = lsn.reshape(NW, KCH, CHUNK)

    splat, slice_k = _sc_kernels()
    sums_flat = splat(pts_pad, ls3d)

    ws = ([*pn_w, *db_down0_w, coarsen0_w, *db_down1_w, coarsen1_w,
           *bottleneck_w, finefy0_w, *db_up0_w, finefy1_w, *db_up1_w,
           stepdown_w[0]])
    a16, b16 = pl.pallas_call(
        _unet_body,
        grid=(GB,),
        in_specs=[pl.BlockSpec((RB, D), lambda i: (i, 0)),
                  pl.BlockSpec((RB, D), lambda i: (GB + i, 0))]
                 + [_wspec(w) for w in ws],
        out_specs=[pl.BlockSpec((RB, D), lambda i: (i, 0)),
                   pl.BlockSpec((RB, D), lambda i: (i, 0))],
        out_shape=[jax.ShapeDtypeStruct((MT, D), f32),
                   jax.ShapeDtypeStruct((MT, D), f32)],
        compiler_params=pltpu.CompilerParams(
            dimension_semantics=("arbitrary",)),
    )(sums_flat, sums_flat, *ws)

    g1, g2 = slice_k(a16, b16, ls3d, lsn3d)

    logits = pl.pallas_call(
        _head_body,
        grid=(GH,),
        in_specs=[pl.BlockSpec((HB, D), lambda i: (i, 0)),
                  pl.BlockSpec((HB, D), lambda i: (i, 0)),
                  _wspec(stepdown_w[1]), _wspec(stepdown_w[2])],
        out_specs=pl.BlockSpec((1, HB, NCLS), lambda i: (0, i, 0)),
        out_shape=jax.ShapeDtypeStruct((1, N, NCLS), f32),
        compiler_params=pltpu.CompilerParams(
            dimension_semantics=("arbitrary",)),
    )(g1, g2, stepdown_w[1], stepdown_w[2])

    return logits


# speedup vs baseline: 5.0831x; 1.1122x over previous
import functools

import jax
import jax.numpy as jnp
from jax import lax
from jax.experimental import pallas as pl
from jax.experimental.pallas import tpu as pltpu
from jax.experimental.pallas import tpu_sc as plsc

N = 100000
M = 50000
NCLS = 21

NC = 2
NS = 16
NW = NC * NS
CHUNK = 128
PPW = 3200
KCH = PPW // CHUNK
NPAD = NW * PPW
D = 16
MT = 51200
MPW = MT // NS
ZROWS = 640

def _splat_body(pts_hbm, idx_hbm, out_hbm, idx_v, rows_v, zbuf, acc):
    cid = lax.axis_index("c")
    sid = lax.axis_index("s")
    wid = sid * NC + cid

    def _zf(i, c):
        zbuf[i, :] = jnp.zeros((16,), jnp.float32)
        return c
    lax.fori_loop(0, ZROWS, _zf, 0)
    for j in range(MPW // ZROWS):
        pltpu.sync_copy(zbuf, acc.at[pl.ds(sid * MPW + j * ZROWS, ZROWS)])
    plsc.subcore_barrier()

    base = wid * PPW
    pltpu.sync_copy(pts_hbm.at[pl.ds(base, PPW)], rows_v)
    pltpu.sync_copy(idx_hbm.at[wid], idx_v)

    def _sc(j, c):
        pltpu.sync_copy(rows_v.at[pl.ds(j * CHUNK, CHUNK)],
                        acc.at[idx_v.at[j]], add=True)
        return c
    lax.fori_loop(0, KCH, _sc, 0)
    plsc.subcore_barrier()

    pltpu.sync_copy(acc.at[pl.ds(sid * MPW, MPW)],
                    out_hbm.at[pl.ds(cid * MT + sid * MPW, MPW)])


@functools.cache
def _sc_kernels():
    mesh = plsc.VectorSubcoreMesh(core_axis_name="c", subcore_axis_name="s",
                                  num_cores=NC, num_subcores=NS)
    params = pltpu.CompilerParams(use_tc_tiling_on_sc=False)
    splat = pl.kernel(
        _splat_body,
        out_type=jax.ShapeDtypeStruct((NC * MT, D), jnp.float32),
        mesh=mesh,
        scratch_types=[
            pltpu.VMEM((KCH, CHUNK), jnp.int32),
            pltpu.VMEM((PPW, D), jnp.float32),
            pltpu.VMEM((ZROWS, D), jnp.float32),
            pltpu.VMEM_SHARED((MT, D), jnp.float32),
        ],
        compiler_params=params,
    )
    slice_k = pl.kernel(
        _slice_body,
        out_type=[jax.ShapeDtypeStruct((NPAD, D), jnp.float32),
                  jax.ShapeDtypeStruct((NPAD, D), jnp.float32)],
        mesh=mesh,
        scratch_types=[
            pltpu.VMEM((KCH, CHUNK), jnp.int32),
            pltpu.VMEM((KCH, CHUNK), jnp.int32),
            pltpu.VMEM((CHUNK, D), jnp.float32),
            pltpu.VMEM((CHUNK, D), jnp.float32),
            pltpu.SemaphoreType.DMA,
            pltpu.SemaphoreType.DMA,
        ],
        compiler_params=params,
    )
    return splat, slice_k


def _slice_body(a_hbm, b_hbm, idx1_hbm, idx2_hbm, g1_hbm, g2_hbm,
                idx1_v, idx2_v, bufa, bufb, sema, semb):
    cid = lax.axis_index("c")
    sid = lax.axis_index("s")
    wid = sid * NC + cid
    base = wid * PPW
    pltpu.sync_copy(idx1_hbm.at[wid], idx1_v)
    pltpu.sync_copy(idx2_hbm.at[wid], idx2_v)

    def _g(j, c):
        ca = pltpu.async_copy(a_hbm.at[idx1_v.at[j]], bufa, sema)
        cb = pltpu.async_copy(b_hbm.at[idx2_v.at[j]], bufb, semb)
        ca.wait()
        cb.wait()
        pltpu.sync_copy(bufa, g1_hbm.at[pl.ds(base + j * CHUNK, CHUNK)])
        pltpu.sync_copy(bufb, g2_hbm.at[pl.ds(base + j * CHUNK, CHUNK)])
        return c
    lax.fori_loop(0, KCH, _g, 0)


RB = 6400
GB = MT // RB


def _relu(x):
    return jnp.maximum(x, 0.0)


def _mm(x, w):
    return jnp.dot(x, w, preferred_element_type=jnp.float32)


def _dense_grow(x, ws):
    for w in ws:
        x = jnp.concatenate([x, _relu(_mm(x, w))], axis=1)
    return x


def _dense_feats(x, ws):
    c0 = x.shape[1]
    return _dense_grow(x, ws)[:, c0:]


def _pairmean(y):
    r, c = y.shape
    y = y.reshape(r // 2, 2, c)
    return (y[:, 0, :] + y[:, 1, :]) * 0.5


def _rep2(y):
    r, c = y.shape
    return jnp.broadcast_to(y[:, None, :], (r, 2, c)).reshape(2 * r, c)


def _unet_body(*refs):
    it = iter(refs)
    sums0_ref = next(it)
    sums1_ref = next(it)
    pn = [next(it) for _ in range(3)]
    d0 = [next(it) for _ in range(4)]
    c0 = next(it)
    d1 = [next(it) for _ in range(5)]
    c1 = next(it)
    bn = [next(it) for _ in range(7)]
    f0 = next(it)
    u0 = [next(it) for _ in range(5)]
    f1 = next(it)
    u1 = [next(it) for _ in range(4)]
    sd0 = next(it)
    a_ref = next(it)
    b_ref = next(it)

    s = sums0_ref[...] + sums1_ref[...]
    cnt = s[:, 6:7]
    dvals = s[:, :6] / jnp.maximum(cnt, 1.0)
    h = dvals
    for w in pn:
        h = _relu(_mm(h, w[...]))
    x = _dense_grow(h, [w[...] for w in d0])
    fv0 = x
    x = _pairmean(_mm(_relu(x), c0[...]))
    x = _dense_grow(x, [w[...] for w in d1])
    fv1 = x
    x = _pairmean(_mm(_relu(x), c1[...]))
    blk = _dense_feats(x, [w[...] for w in bn])
    fine = _rep2(_mm(_relu(blk), f0[...]))
    x = jnp.concatenate([fine, fv1], axis=1)
    blk = _dense_feats(x, [w[...] for w in u0])
    fine = _rep2(_mm(_relu(blk), f1[...]))
    x = jnp.concatenate([fine, fv0], axis=1)
    blk = _dense_feats(x, [w[...] for w in u1])
    w0 = sd0[...]
    a_ref[...] = _mm(blk, w0[:32])
    b_ref[...] = _mm(blk, w0[32:])


HB = 4000
GH = N // HB


def _head_body(g1_ref, g2_ref, w1_ref, w2_ref, out_ref):
    h = _relu(g1_ref[...] + g2_ref[...])
    h = _relu(_mm(h, w1_ref[...]))
    lg = _mm(h, w2_ref[...])
    m = jnp.max(lg, axis=1, keepdims=True)
    e = jnp.exp(lg - m)
    out_ref[0] = lg - m - jnp.log(jnp.sum(e, axis=1, keepdims=True))


def _wspec(w):
    return pl.BlockSpec(w.shape, lambda i, _nd=w.ndim: (0,) * _nd)


def kernel(ls, positions, values, pn_w, db_down0_w, coarsen0_w, db_down1_w,
           coarsen1_w, bottleneck_w, finefy0_w, db_up0_w, finefy1_w,
           db_up1_w, stepdown_w):
    f32 = jnp.float32
    pts = jnp.concatenate(
        [positions[0], values[0],
         jnp.ones((N, 1), f32), jnp.zeros((N, D - 7), f32)], axis=1)
    pts_pad = jnp.concatenate([pts, jnp.zeros((NPAD - N, D), f32)], axis=0)
    ls_pad = jnp.concatenate([ls, jnp.zeros((NPAD - N,), jnp.int32)])
    ls3d = ls_pad.reshape(NW, KCH, CHUNK)
    lsn = (ls_pad + 1) % M
    lsn3d = lsn.reshape(NW, KCH, CHUNK)

    splat, slice_k = _sc_kernels()
    sums_flat = splat(pts_pad, ls3d)

    ws = ([*pn_w, *db_down0_w, coarsen0_w, *db_down1_w, coarsen1_w,
           *bottleneck_w, finefy0_w, *db_up0_w, finefy1_w, *db_up1_w,
           stepdown_w[0]])
    a16, b16 = pl.pallas_call(
        _unet_body,
        grid=(GB,),
        in_specs=[pl.BlockSpec((RB, D), lambda i: (i, 0)),
                  pl.BlockSpec((RB, D), lambda i: (GB + i, 0))]
                 + [_wspec(w) for w in ws],
        out_specs=[pl.BlockSpec((RB, D), lambda i: (i, 0)),
                   pl.BlockSpec((RB, D), lambda i: (i, 0))],
        out_shape=[jax.ShapeDtypeStruct((MT, D), f32),
                   jax.ShapeDtypeStruct((MT, D), f32)],
        compiler_params=pltpu.CompilerParams(
            dimension_semantics=("arbitrary",)),
    )(sums_flat, sums_flat, *ws)

    g1, g2 = slice_k(a16, b16, ls3d, lsn3d)

    logits = pl.pallas_call(
        _head_body,
        grid=(GH,),
        in_specs=[pl.BlockSpec((HB, D), lambda i: (i, 0)),
                  pl.BlockSpec((HB, D), lambda i: (i, 0)),
                  _wspec(stepdown_w[1]), _wspec(stepdown_w[2])],
        out_specs=pl.BlockSpec((1, HB, NCLS), lambda i: (0, i, 0)),
        out_shape=jax.ShapeDtypeStruct((1, N, NCLS), f32),
        compiler_params=pltpu.CompilerParams(
            dimension_semantics=("arbitrary",)),
    )(g1, g2, stepdown_w[1], stepdown_w[2])

    return logits
